# baseline (speedup 1.0000x reference)
import functools

import jax
import jax.numpy as jnp
from jax import lax
from jax.experimental import pallas as pl
from jax.experimental.pallas import tpu as pltpu
from jax.experimental.pallas import tpu_sc as plsc

NUM_USERS = 1000000
EMBED_DIM = 32
BATCH = 16384

_UPAD = 1000064
_NGRAN = _UPAD // 128
_NROWS = EMBED_DIM * _NGRAN

_info = plsc.get_sparse_core_info()
_NC, _NS = _info.num_cores, _info.num_subcores
_NW = _NC * _NS
_B_PER_W = BATCH // _NW
_NG = _B_PER_W // 16

_mesh = plsc.VectorSubcoreMesh(core_axis_name="c", subcore_axis_name="s")


def _iota16():
    return lax.iota(jnp.int32, 16)


@functools.partial(
    pl.kernel,
    mesh=_mesh,
    compiler_params=pltpu.CompilerParams(needs_layout_passes=False),
    out_type=jax.ShapeDtypeStruct((EMBED_DIM, BATCH), jnp.float32),
    scratch_types=[
        pltpu.VMEM((_B_PER_W,), jnp.int32),
        pltpu.VMEM((128,), jnp.int32),
        pltpu.VMEM((128,), jnp.int32),
        pltpu.VMEM((128,), jnp.int32),
        pltpu.VMEM((128,), jnp.int32),
        pltpu.VMEM((256, 128), jnp.float32),
        pltpu.VMEM((256, 128), jnp.float32),
        pltpu.VMEM((EMBED_DIM, _B_PER_W), jnp.float32),
        pltpu.SemaphoreType.DMA,
        pltpu.SemaphoreType.DMA,
    ],
)
def _lookup(idx_hbm, tview_hbm, out_hbm,
            idx_v, rla0, rlb0, rla1, rlb1, gbuf0, gbuf1, outb, s0, s1):
    cid = lax.axis_index("c")
    sid = lax.axis_index("s")
    wid = sid * _NC + cid
    base = wid * _B_PER_W
    pltpu.sync_copy(idx_hbm.at[pl.ds(base, _B_PER_W)], idx_v)

    def build_and_fire(unit, rla, rlb, gbuf, sem):
        g = unit >> 1
        h = unit & 1
        guv = idx_v[pl.ds(g * 16, 16)] >> 7
        for d in range(8):
            rla[pl.ds(d * 16, 16)] = guv + (16 * h + d) * _NGRAN
            rlb[pl.ds(d * 16, 16)] = guv + (16 * h + 8 + d) * _NGRAN
        pltpu.async_copy(tview_hbm.at[rla], gbuf.at[pl.ds(0, 128)], sem)
        pltpu.async_copy(tview_hbm.at[rlb], gbuf.at[pl.ds(128, 128)], sem)

    def consume(unit, gbuf, sem):
        pltpu.make_async_copy(tview_hbm.at[pl.ds(0, 256)], gbuf, sem).wait()
        g = unit >> 1
        h = unit & 1
        lane = idx_v[pl.ds(g * 16, 16)] & 127
        for d in range(16):
            vals = plsc.load_gather(gbuf, [d * 16 + _iota16(), lane])
            outb[16 * h + d, pl.ds(pl.multiple_of(g * 16, 16), 16)] = vals

    n_units = 2 * _NG
    build_and_fire(jnp.int32(0), rla0, rlb0, gbuf0, s0)
    build_and_fire(jnp.int32(1), rla1, rlb1, gbuf1, s1)

    def pair_body(p, carry):
        u0 = 2 * p
        consume(u0, gbuf0, s0)

        @pl.when(u0 + 2 < n_units)
        def _f0():
            build_and_fire(u0 + 2, rla0, rlb0, gbuf0, s0)

        consume(u0 + 1, gbuf1, s1)

        @pl.when(u0 + 3 < n_units)
        def _f1():
            build_and_fire(u0 + 3, rla1, rlb1, gbuf1, s1)

        return carry

    lax.fori_loop(0, n_units // 2, pair_body, jnp.int32(0))

    pltpu.sync_copy(
        outb, out_hbm.at[:, pl.ds(pl.multiple_of(base, 128), _B_PER_W)]
    )


def kernel(user_id, table):
    tview = jnp.pad(table.T, ((0, 0), (0, _UPAD - (NUM_USERS + 1)))).reshape(
        _NROWS, 128
    )
    out_t = _lookup(user_id.astype(jnp.int32), tview)
    return out_t.T


# speedup vs baseline: 1.7488x; 1.7488x over previous
import functools

import jax
import jax.numpy as jnp
from jax import lax
from jax.experimental import pallas as pl
from jax.experimental.pallas import tpu as pltpu
from jax.experimental.pallas import tpu_sc as plsc

NUM_USERS = 1000000
EMBED_DIM = 32
BATCH = 16384

_UPAD = 1000064
_NGRAN = _UPAD // 128
_NROWS = EMBED_DIM * _NGRAN

_info = plsc.get_sparse_core_info()
_NC, _NS = _info.num_cores, _info.num_subcores
_NW = _NC * _NS
_B_PER_W = BATCH // _NW
_NG = _B_PER_W // 16

_mesh = plsc.VectorSubcoreMesh(core_axis_name="c", subcore_axis_name="s")


def _iota16():
    return lax.iota(jnp.int32, 16)


@functools.partial(
    pl.kernel,
    mesh=_mesh,
    compiler_params=pltpu.CompilerParams(needs_layout_passes=False),
    out_type=jax.ShapeDtypeStruct((EMBED_DIM, BATCH), jnp.float32),
    scratch_types=[
        pltpu.VMEM((_B_PER_W,), jnp.int32),
        pltpu.VMEM((128,), jnp.int32),
        pltpu.VMEM((128,), jnp.int32),
        pltpu.VMEM((128,), jnp.int32),
        pltpu.VMEM((128,), jnp.int32),
        pltpu.VMEM((256, 128), jnp.float32),
        pltpu.VMEM((256, 128), jnp.float32),
        pltpu.VMEM((EMBED_DIM, _B_PER_W), jnp.float32),
        pltpu.SemaphoreType.DMA,
        pltpu.SemaphoreType.DMA,
    ],
)
def _lookup(idx_hbm, tview_hbm, out_hbm,
            idx_v, rla0, rlb0, rla1, rlb1, gbuf0, gbuf1, outb, s0, s1):
    cid = lax.axis_index("c")
    sid = lax.axis_index("s")
    wid = sid * _NC + cid
    base = wid * _B_PER_W
    pltpu.sync_copy(idx_hbm.at[pl.ds(base, _B_PER_W)], idx_v)

    def build_and_fire(unit, rla, rlb, gbuf, sem):
        g = unit >> 1
        h = unit & 1
        guv8 = (idx_v[pl.ds(g * 16, 16)] >> 7) << 3
        for d in range(8):
            rla[pl.ds(d * 16, 16)] = guv8 + ((2 * h) * 8 * _NGRAN + d)
            rlb[pl.ds(d * 16, 16)] = guv8 + ((2 * h + 1) * 8 * _NGRAN + d)
        pltpu.async_copy(tview_hbm.at[rla], gbuf.at[pl.ds(0, 128)], sem)
        pltpu.async_copy(tview_hbm.at[rlb], gbuf.at[pl.ds(128, 128)], sem)

    def consume(unit, gbuf, sem):
        pltpu.make_async_copy(tview_hbm.at[pl.ds(0, 256)], gbuf, sem).wait()
        g = unit >> 1
        h = unit & 1
        lane = idx_v[pl.ds(g * 16, 16)] & 127
        for d in range(16):
            vals = plsc.load_gather(gbuf, [d * 16 + _iota16(), lane])
            outb[16 * h + d, pl.ds(pl.multiple_of(g * 16, 16), 16)] = vals

    n_units = 2 * _NG
    build_and_fire(jnp.int32(0), rla0, rlb0, gbuf0, s0)
    build_and_fire(jnp.int32(1), rla1, rlb1, gbuf1, s1)

    def pair_body(p, carry):
        u0 = 2 * p
        consume(u0, gbuf0, s0)

        @pl.when(u0 + 2 < n_units)
        def _f0():
            build_and_fire(u0 + 2, rla0, rlb0, gbuf0, s0)

        consume(u0 + 1, gbuf1, s1)

        @pl.when(u0 + 3 < n_units)
        def _f1():
            build_and_fire(u0 + 3, rla1, rlb1, gbuf1, s1)

        return carry

    lax.fori_loop(0, n_units // 2, pair_body, jnp.int32(0))

    pltpu.sync_copy(
        outb, out_hbm.at[:, pl.ds(pl.multiple_of(base, 128), _B_PER_W)]
    )


def kernel(user_id, table):
    tview = (
        jnp.pad(table, ((0, _UPAD - (NUM_USERS + 1)), (0, 0)))
        .reshape(_NGRAN, 128, EMBED_DIM // 8, 8)
        .transpose(2, 0, 3, 1)
        .reshape(_NROWS, 128)
    )
    out_t = _lookup(user_id.astype(jnp.int32), tview)
    return out_t.T
